# baseline (speedup 1.0000x reference)
import jax
import jax.numpy as jnp
from jax import lax
from jax.experimental import pallas as pl
from jax.experimental.pallas import tpu as pltpu
from jax.experimental.pallas import tpu_sc as plsc

N = 10000
F = 128
NEG_SLOPE = 0.2

NC = 2
NS = 16
L = 16
NW = NC * NS

NPAD = 10112
RPT = NPAD // NS
CH = 64
GB = 8
NBLK = 21
NCHUNK = GB * NBLK
EW = CH * NCHUNK
EPAD = EW * NW
E2 = 320000 + N

TCB = NPAD

_RPT_CHUNKS = []
_off = 0
while _off < RPT:
    _ln = min(64, RPT - _off)
    _RPT_CHUNKS.append((_off, _ln))
    _off += _ln


def _i32(v):
    return jnp.int32(v)


def _tc_pre_body(x_ref, w_ref, am_ref, h_ref, s_ref):
    h = jnp.dot(x_ref[...], w_ref[...], preferred_element_type=jnp.float32)
    h_ref[...] = h
    s_ref[...] = jnp.dot(h, am_ref[...], preferred_element_type=jnp.float32)


_tc_pre = pl.pallas_call(
    _tc_pre_body,
    grid=(NPAD // TCB,),
    in_specs=[
        pl.BlockSpec((TCB, F), lambda i: (i, i * 0)),
        pl.BlockSpec((F, F), lambda i: (i * 0, i * 0)),
        pl.BlockSpec((F, F), lambda i: (i * 0, i * 0)),
    ],
    out_specs=[
        pl.BlockSpec((TCB, F), lambda i: (i, i * 0)),
        pl.BlockSpec((TCB, F), lambda i: (i, i * 0)),
    ],
    out_shape=[
        jax.ShapeDtypeStruct((NPAD, F), jnp.float32),
        jax.ShapeDtypeStruct((NPAD, F), jnp.float32),
    ],
)


def _sc_body(src_hbm, dst_hbm, h_hbm, ai_hbm, aj_hbm,
             out_hbm, den_hbm,
             ai_v, aj_v, den_v, src_v, dst_v, w_v,
             rows_a, rows_b, acc_sh, sem_a, sem_b, sem_sa, sem_sb):
    cid = lax.axis_index("c").astype(jnp.int32)
    sid = lax.axis_index("s").astype(jnp.int32)
    wid = sid * _i32(NC) + cid

    pltpu.sync_copy(ai_hbm, ai_v)
    pltpu.sync_copy(aj_hbm, aj_v)

    zero16 = jnp.zeros((L,), jnp.float32)

    def _zero_den(i, carry):
        den_v[pl.ds(i * _i32(L), L)] = zero16
        return carry
    lax.fori_loop(_i32(0), _i32(NPAD // L), _zero_den, _i32(0))

    def _zero_rows(r, carry):
        for c in range(F // L):
            rows_a[r, pl.ds(c * L, L)] = zero16
        return carry
    lax.fori_loop(_i32(0), _i32(CH), _zero_rows, _i32(0))

    row0 = sid * _i32(RPT)
    for off, ln in _RPT_CHUNKS:
        pltpu.sync_copy(rows_a.at[pl.ds(0, ln)],
                        acc_sh.at[pl.ds(row0 + _i32(off), ln)])
    plsc.subcore_barrier()

    def _gather(c, rows, sem):
        pltpu.async_copy(h_hbm.at[src_v.at[c]], rows, sem)

    def _wait(c, rows, sem):
        pltpu.make_async_copy(h_hbm.at[src_v.at[c]], rows, sem).wait()

    def _scale(rows, c):
        def body(r, carry):
            wb = plsc.load_gather(
                w_v, [jnp.full((L,), c, jnp.int32),
                      jnp.full((L,), r, jnp.int32)])
            for q in range(F // L):
                rows[r, pl.ds(q * L, L)] = rows[r, pl.ds(q * L, L)] * wb
            return carry
        lax.fori_loop(_i32(0), _i32(CH), body, _i32(0))

    def _scatter(rows, c, sem):
        pltpu.async_copy(rows, acc_sh.at[dst_v.at[c]], sem, add=True)

    def _wait_s(rows, c, sem):
        pltpu.make_async_copy(rows, acc_sh.at[dst_v.at[c]], sem).wait()

    def _scale_half(rows, c, half):
        def body(r, carry):
            wb = plsc.load_gather(
                w_v, [jnp.full((L,), c, jnp.int32),
                      jnp.full((L,), r, jnp.int32)])
            for q in range(F // L):
                rows[r, pl.ds(q * L, L)] = rows[r, pl.ds(q * L, L)] * wb
            return carry
        lax.fori_loop(_i32(half * (CH // 2)), _i32((half + 1) * (CH // 2)),
                      body, _i32(0))

    def _block(b, carry):
        @pl.when(b > 0)
        def _():
            _wait_s(rows_a, _i32(GB - 2), sem_sa)
            _wait_s(rows_b, _i32(GB - 1), sem_sb)
        pltpu.sync_copy(src_hbm.at[wid, b], src_v)
        pltpu.sync_copy(dst_hbm.at[wid, b], dst_v)

        _gather(_i32(0), rows_a, sem_a)
        _gather(_i32(1), rows_b, sem_b)

        def _alpha(j, c2):
            r = lax.shift_right_logical(j, _i32(2))
            col = lax.shift_left(jnp.bitwise_and(j, _i32(3)), _i32(4))
            s16 = src_v[r, pl.ds(col, L)]
            d16 = dst_v[r, pl.ds(col, L)]
            a = plsc.load_gather(ai_v, [d16]) + plsc.load_gather(aj_v, [s16])
            a = jnp.where(a >= 0, a, a * NEG_SLOPE)
            wv = jnp.exp(a)
            w_v[r, pl.ds(col, L)] = wv
            plsc.addupdate_scatter(den_v, [d16], wv)
            return c2
        lax.fori_loop(_i32(0), _i32(GB * CH // L), _alpha, _i32(0))

        for c in range(GB):
            rows = rows_a if c % 2 == 0 else rows_b
            sem = sem_a if c % 2 == 0 else sem_b
            ssem = sem_sa if c % 2 == 0 else sem_sb
            _wait(_i32(c), rows, sem)
            _scale_half(rows, _i32(c), 0)
            if 2 <= c + 1 < GB:
                prows = rows_b if c % 2 == 0 else rows_a
                pg = sem_b if c % 2 == 0 else sem_a
                ps = sem_sb if c % 2 == 0 else sem_sa
                _wait_s(prows, _i32(c - 1), ps)
                _gather(_i32(c + 1), prows, pg)
            _scale_half(rows, _i32(c), 1)
            _scatter(rows, _i32(c), ssem)
        return carry
    lax.fori_loop(_i32(0), _i32(NBLK), _block, _i32(0))

    _wait_s(rows_a, _i32(GB - 2), sem_sa)
    _wait_s(rows_b, _i32(GB - 1), sem_sb)
    pltpu.sync_copy(den_v, den_hbm.at[wid])
    plsc.subcore_barrier()
    for off, ln in _RPT_CHUNKS:
        sl = pl.ds(row0 + _i32(off), ln)
        pltpu.sync_copy(acc_sh.at[sl], out_hbm.at[cid, sl])


_sc_call = pl.kernel(
    _sc_body,
    out_type=[
        jax.ShapeDtypeStruct((NC, NPAD, F), jnp.float32),
        jax.ShapeDtypeStruct((NW, NPAD), jnp.float32),
    ],
    mesh=plsc.VectorSubcoreMesh(
        core_axis_name="c", subcore_axis_name="s",
        num_cores=NC, num_subcores=NS),
    scratch_types=[
        pltpu.VMEM((NPAD,), jnp.float32),
        pltpu.VMEM((NPAD,), jnp.float32),
        pltpu.VMEM((NPAD,), jnp.float32),
        pltpu.VMEM((GB, CH), jnp.int32),
        pltpu.VMEM((GB, CH), jnp.int32),
        pltpu.VMEM((GB, CH), jnp.float32),
        pltpu.VMEM((CH, F), jnp.float32),
        pltpu.VMEM((CH, F), jnp.float32),
        pltpu.VMEM_SHARED((NPAD, F), jnp.float32),
        pltpu.SemaphoreType.DMA,
        pltpu.SemaphoreType.DMA,
        pltpu.SemaphoreType.DMA,
        pltpu.SemaphoreType.DMA,
    ],
    compiler_params=pltpu.CompilerParams(needs_layout_passes=False),
)


def _tc_post_body(p0_ref, p1_ref, den_ref, b_ref, o_ref):
    den = jnp.sum(den_ref[...], axis=0) + jnp.float32(1e-16)
    o_ref[...] = (p0_ref[...] + p1_ref[...]) / den[:, None] + b_ref[...]


_tc_post = pl.pallas_call(
    _tc_post_body,
    grid=(NPAD // TCB,),
    in_specs=[
        pl.BlockSpec((TCB, F), lambda i: (i, i * 0)),
        pl.BlockSpec((TCB, F), lambda i: (i, i * 0)),
        pl.BlockSpec((NW, TCB), lambda i: (i * 0, i)),
        pl.BlockSpec((1, F), lambda i: (i * 0, i * 0)),
    ],
    out_specs=pl.BlockSpec((TCB, F), lambda i: (i, i * 0)),
    out_shape=jax.ShapeDtypeStruct((NPAD, F), jnp.float32),
)


def kernel(x, edge_index, weight, att, bias):
    x = x.astype(jnp.float32)
    weight = weight.astype(jnp.float32)
    att_f = att.astype(jnp.float32).reshape(2 * F)
    bias = bias.astype(jnp.float32)

    src = edge_index[0].astype(jnp.int32)
    dst = edge_index[1].astype(jnp.int32)
    dst = jnp.where(src == dst, jnp.int32(N), dst)
    loops = jnp.arange(N, dtype=jnp.int32)
    pad_src = jnp.arange(EPAD - E2, dtype=jnp.int32) % jnp.int32(N)
    pad_dst = jnp.int32(N + 16) + (jnp.arange(EPAD - E2, dtype=jnp.int32)
                                   % jnp.int32(NPAD - N - 16))
    src2 = jnp.concatenate([src, loops, pad_src]).reshape(NW, NBLK, GB, CH)
    dst2 = jnp.concatenate([dst, loops, pad_dst]).reshape(NW, NBLK, GB, CH)

    x_pad = jnp.zeros((NPAD, F), jnp.float32).at[:N].set(x)
    attmat = (jnp.zeros((F, F), jnp.float32)
              .at[:, 0].set(att_f[:F])
              .at[:, 1].set(att_f[F:]))

    h, s = _tc_pre(x_pad, weight, attmat)
    ai = s[:, 0] + 0.0
    aj = s[:, 1] + 0.0

    partials, dens = _sc_call(src2, dst2, h, ai, aj)

    out = _tc_post(partials[0], partials[1], dens, bias.reshape(1, F))
    return out[:N]


# speedup vs baseline: 1.0888x; 1.0888x over previous
import jax
import jax.numpy as jnp
from jax import lax
from jax.experimental import pallas as pl
from jax.experimental.pallas import tpu as pltpu
from jax.experimental.pallas import tpu_sc as plsc

N = 10000
F = 128
NEG_SLOPE = 0.2

NC = 2
NS = 16
L = 16
NW = NC * NS

NPAD = 10112
RPT = NPAD // NS
CH = 64
GB = 8
NBLK = 21
NCHUNK = GB * NBLK
EW = CH * NCHUNK
EPAD = EW * NW
E2 = 320000 + N

TCB = NPAD

_RPT_CHUNKS = []
_off = 0
while _off < RPT:
    _ln = min(64, RPT - _off)
    _RPT_CHUNKS.append((_off, _ln))
    _off += _ln


def _i32(v):
    return jnp.int32(v)


def _tc_pre_body(x_ref, w_ref, am_ref, h_ref, s_ref):
    h = jnp.dot(x_ref[...], w_ref[...], preferred_element_type=jnp.float32)
    h_ref[...] = h
    s_ref[...] = jnp.dot(h, am_ref[...], preferred_element_type=jnp.float32)


_tc_pre = pl.pallas_call(
    _tc_pre_body,
    grid=(NPAD // TCB,),
    in_specs=[
        pl.BlockSpec((TCB, F), lambda i: (i, i * 0)),
        pl.BlockSpec((F, F), lambda i: (i * 0, i * 0)),
        pl.BlockSpec((F, F), lambda i: (i * 0, i * 0)),
    ],
    out_specs=[
        pl.BlockSpec((TCB, F), lambda i: (i, i * 0)),
        pl.BlockSpec((TCB, F), lambda i: (i, i * 0)),
    ],
    out_shape=[
        jax.ShapeDtypeStruct((NPAD, F), jnp.float32),
        jax.ShapeDtypeStruct((NPAD, F), jnp.float32),
    ],
)


def _sc_body(src_hbm, dst_hbm, h_hbm, ai_hbm, aj_hbm,
             out_hbm, den_hbm,
             ai_v, aj_v, den_v, src_v, dst_v, w_v,
             rows_a, rows_b, acc_sh, sem_a, sem_b):
    cid = lax.axis_index("c").astype(jnp.int32)
    sid = lax.axis_index("s").astype(jnp.int32)
    wid = sid * _i32(NC) + cid

    pltpu.sync_copy(ai_hbm, ai_v)
    pltpu.sync_copy(aj_hbm, aj_v)

    zero16 = jnp.zeros((L,), jnp.float32)

    def _zero_den(i, carry):
        den_v[pl.ds(i * _i32(L), L)] = zero16
        return carry
    lax.fori_loop(_i32(0), _i32(NPAD // L), _zero_den, _i32(0))

    def _zero_rows(r, carry):
        for c in range(F // L):
            rows_a[r, pl.ds(c * L, L)] = zero16
        return carry
    lax.fori_loop(_i32(0), _i32(CH), _zero_rows, _i32(0))

    row0 = sid * _i32(RPT)
    for off, ln in _RPT_CHUNKS:
        pltpu.sync_copy(rows_a.at[pl.ds(0, ln)],
                        acc_sh.at[pl.ds(row0 + _i32(off), ln)])
    plsc.subcore_barrier()

    def _gather(c, rows, sem):
        pltpu.async_copy(h_hbm.at[src_v.at[c]], rows, sem)

    def _wait(c, rows, sem):
        pltpu.make_async_copy(h_hbm.at[src_v.at[c]], rows, sem).wait()

    def _scale(rows, c):
        def body(r, carry):
            wb = plsc.load_gather(
                w_v, [jnp.full((L,), c, jnp.int32),
                      jnp.full((L,), r, jnp.int32)])
            for q in range(F // L):
                rows[r, pl.ds(q * L, L)] = rows[r, pl.ds(q * L, L)] * wb
            return carry
        lax.fori_loop(_i32(0), _i32(CH), body, _i32(0))

    def _scatter(rows, c):
        pltpu.sync_copy(rows, acc_sh.at[dst_v.at[c]], add=True)

    def _block(b, carry):
        pltpu.sync_copy(src_hbm.at[wid, b], src_v)
        pltpu.sync_copy(dst_hbm.at[wid, b], dst_v)

        _gather(_i32(0), rows_a, sem_a)
        _gather(_i32(1), rows_b, sem_b)

        def _alpha(j, c2):
            r = lax.shift_right_logical(j, _i32(2))
            col = lax.shift_left(jnp.bitwise_and(j, _i32(3)), _i32(4))
            s16 = src_v[r, pl.ds(col, L)]
            d16 = dst_v[r, pl.ds(col, L)]
            a = plsc.load_gather(ai_v, [d16]) + plsc.load_gather(aj_v, [s16])
            a = jnp.where(a >= 0, a, a * NEG_SLOPE)
            wv = jnp.exp(a)
            w_v[r, pl.ds(col, L)] = wv
            plsc.addupdate_scatter(den_v, [d16], wv)
            return c2
        lax.fori_loop(_i32(0), _i32(GB * CH // L), _alpha, _i32(0))

        for c in range(GB):
            rows = rows_a if c % 2 == 0 else rows_b
            sem = sem_a if c % 2 == 0 else sem_b
            _wait(_i32(c), rows, sem)
            _scale(rows, _i32(c))
            _scatter(rows, _i32(c))
            if c + 2 < GB:
                _gather(_i32(c + 2), rows, sem)
        return carry
    lax.fori_loop(_i32(0), _i32(NBLK), _block, _i32(0))

    pltpu.sync_copy(den_v, den_hbm.at[wid])
    plsc.subcore_barrier()
    for off, ln in _RPT_CHUNKS:
        sl = pl.ds(row0 + _i32(off), ln)
        pltpu.sync_copy(acc_sh.at[sl], out_hbm.at[cid, sl])


_sc_call = pl.kernel(
    _sc_body,
    out_type=[
        jax.ShapeDtypeStruct((NC, NPAD, F), jnp.float32),
        jax.ShapeDtypeStruct((NW, NPAD), jnp.float32),
    ],
    mesh=plsc.VectorSubcoreMesh(
        core_axis_name="c", subcore_axis_name="s",
        num_cores=NC, num_subcores=NS),
    scratch_types=[
        pltpu.VMEM((NPAD,), jnp.float32),
        pltpu.VMEM((NPAD,), jnp.float32),
        pltpu.VMEM((NPAD,), jnp.float32),
        pltpu.VMEM((GB, CH), jnp.int32),
        pltpu.VMEM((GB, CH), jnp.int32),
        pltpu.VMEM((GB, CH), jnp.float32),
        pltpu.VMEM((CH, F), jnp.float32),
        pltpu.VMEM((CH, F), jnp.float32),
        pltpu.VMEM_SHARED((NPAD, F), jnp.float32),
        pltpu.SemaphoreType.DMA,
        pltpu.SemaphoreType.DMA,
    ],
    compiler_params=pltpu.CompilerParams(needs_layout_passes=False),
)


def _tc_post_body(p0_ref, p1_ref, den_ref, b_ref, o_ref):
    den = jnp.sum(den_ref[...], axis=0) + jnp.float32(1e-16)
    o_ref[...] = (p0_ref[...] + p1_ref[...]) / den[:, None] + b_ref[...]


_tc_post = pl.pallas_call(
    _tc_post_body,
    grid=(NPAD // TCB,),
    in_specs=[
        pl.BlockSpec((TCB, F), lambda i: (i, i * 0)),
        pl.BlockSpec((TCB, F), lambda i: (i, i * 0)),
        pl.BlockSpec((NW, TCB), lambda i: (i * 0, i)),
        pl.BlockSpec((1, F), lambda i: (i * 0, i * 0)),
    ],
    out_specs=pl.BlockSpec((TCB, F), lambda i: (i, i * 0)),
    out_shape=jax.ShapeDtypeStruct((NPAD, F), jnp.float32),
)


def kernel(x, edge_index, weight, att, bias):
    x = x.astype(jnp.float32)
    weight = weight.astype(jnp.float32)
    att_f = att.astype(jnp.float32).reshape(2 * F)
    bias = bias.astype(jnp.float32)

    src = edge_index[0].astype(jnp.int32)
    dst = edge_index[1].astype(jnp.int32)
    dst = jnp.where(src == dst, jnp.int32(N), dst)
    loops = jnp.arange(N, dtype=jnp.int32)
    pad_src = jnp.arange(EPAD - E2, dtype=jnp.int32) % jnp.int32(N)
    pad_dst = jnp.int32(N + 16) + (jnp.arange(EPAD - E2, dtype=jnp.int32)
                                   % jnp.int32(NPAD - N - 16))
    src2 = jnp.concatenate([src, loops, pad_src]).reshape(NW, NBLK, GB, CH)
    dst2 = jnp.concatenate([dst, loops, pad_dst]).reshape(NW, NBLK, GB, CH)

    x_pad = jnp.zeros((NPAD, F), jnp.float32).at[:N].set(x)
    attmat = (jnp.zeros((F, F), jnp.float32)
              .at[:, 0].set(att_f[:F])
              .at[:, 1].set(att_f[F:]))

    h, s = _tc_pre(x_pad, weight, attmat)
    ai = s[:, 0] + 0.0
    aj = s[:, 1] + 0.0

    partials, dens = _sc_call(src2, dst2, h, ai, aj)

    out = _tc_post(partials[0], partials[1], dens, bias.reshape(1, F))
    return out[:N]


# speedup vs baseline: 1.1671x; 1.0719x over previous
import jax
import jax.numpy as jnp
from jax import lax
from jax.experimental import pallas as pl
from jax.experimental.pallas import tpu as pltpu
from jax.experimental.pallas import tpu_sc as plsc

N = 10000
F = 128
NEG_SLOPE = 0.2

NC = 2
NS = 16
L = 16
NW = NC * NS

NPAD = 10112
RPT = NPAD // NS
CH = 64
GB = 8
NBLK = 21
NCHUNK = GB * NBLK
EW = CH * NCHUNK
EPAD = EW * NW
E2 = 320000 + N

TCB = NPAD

_RPT_CHUNKS = []
_off = 0
while _off < RPT:
    _ln = min(64, RPT - _off)
    _RPT_CHUNKS.append((_off, _ln))
    _off += _ln


def _i32(v):
    return jnp.int32(v)


def _tc_pre_body(x_ref, w_ref, am_ref, h_ref, s_ref):
    h = jnp.dot(x_ref[...], w_ref[...], preferred_element_type=jnp.float32)
    h_ref[...] = h
    s_ref[...] = jnp.dot(h, am_ref[...], preferred_element_type=jnp.float32)


_tc_pre = pl.pallas_call(
    _tc_pre_body,
    grid=(NPAD // TCB,),
    in_specs=[
        pl.BlockSpec((TCB, F), lambda i: (i, i * 0)),
        pl.BlockSpec((F, F), lambda i: (i * 0, i * 0)),
        pl.BlockSpec((F, F), lambda i: (i * 0, i * 0)),
    ],
    out_specs=[
        pl.BlockSpec((TCB, F), lambda i: (i, i * 0)),
        pl.BlockSpec((TCB, F), lambda i: (i, i * 0)),
    ],
    out_shape=[
        jax.ShapeDtypeStruct((NPAD, F), jnp.float32),
        jax.ShapeDtypeStruct((NPAD, F), jnp.float32),
    ],
)


def _sc_body(src_hbm, dst_hbm, h_hbm, ai_hbm, aj_hbm,
             out_hbm, den_hbm,
             ai_v, aj_v, den_v, src_v, dst_v, w_v,
             rows_a, rows_b, acc_sh, sem_a, sem_b):
    cid = lax.axis_index("c").astype(jnp.int32)
    sid = lax.axis_index("s").astype(jnp.int32)
    wid = sid * _i32(NC) + cid

    pltpu.sync_copy(ai_hbm, ai_v)
    pltpu.sync_copy(aj_hbm, aj_v)

    zero16 = jnp.zeros((L,), jnp.float32)

    def _zero_den(i, carry):
        den_v[pl.ds(i * _i32(L), L)] = zero16
        return carry
    lax.fori_loop(_i32(0), _i32(NPAD // L), _zero_den, _i32(0))

    def _zero_rows(r, carry):
        for c in range(F // L):
            rows_a[r, pl.ds(c * L, L)] = zero16
        return carry
    lax.fori_loop(_i32(0), _i32(CH), _zero_rows, _i32(0))

    row0 = sid * _i32(RPT)
    for off, ln in _RPT_CHUNKS:
        pltpu.sync_copy(rows_a.at[pl.ds(0, ln)],
                        acc_sh.at[pl.ds(row0 + _i32(off), ln)])
    plsc.subcore_barrier()

    def _gather(c, rows, sem):
        pltpu.async_copy(h_hbm.at[src_v.at[c]], rows, sem)

    def _wait(c, rows, sem):
        pltpu.make_async_copy(h_hbm.at[src_v.at[c]], rows, sem).wait()

    def _scale(rows, c):
        def body(r2, carry):
            r0 = r2 * _i32(2)
            r1 = r0 + _i32(1)
            cidx = jnp.full((L,), c, jnp.int32)
            wb0 = plsc.load_gather(w_v, [cidx, jnp.full((L,), r0, jnp.int32)])
            wb1 = plsc.load_gather(w_v, [cidx, jnp.full((L,), r1, jnp.int32)])
            for q in range(F // L):
                rows[r0, pl.ds(q * L, L)] = rows[r0, pl.ds(q * L, L)] * wb0
                rows[r1, pl.ds(q * L, L)] = rows[r1, pl.ds(q * L, L)] * wb1
            return carry
        lax.fori_loop(_i32(0), _i32(CH // 2), body, _i32(0))

    def _scatter(rows, c):
        pltpu.sync_copy(rows, acc_sh.at[dst_v.at[c]], add=True)

    def _block(b, carry):
        pltpu.sync_copy(src_hbm.at[wid, b], src_v)
        pltpu.sync_copy(dst_hbm.at[wid, b], dst_v)

        _gather(_i32(0), rows_a, sem_a)
        _gather(_i32(1), rows_b, sem_b)

        def _alpha(j, c2):
            r = lax.shift_right_logical(j, _i32(2))
            col = lax.shift_left(jnp.bitwise_and(j, _i32(3)), _i32(4))
            s16 = src_v[r, pl.ds(col, L)]
            d16 = dst_v[r, pl.ds(col, L)]
            a = plsc.load_gather(ai_v, [d16]) + plsc.load_gather(aj_v, [s16])
            a = jnp.where(a >= 0, a, a * NEG_SLOPE)
            wv = jnp.exp(a)
            w_v[r, pl.ds(col, L)] = wv
            plsc.addupdate_scatter(den_v, [d16], wv)
            return c2
        lax.fori_loop(_i32(0), _i32(GB * CH // L), _alpha, _i32(0))

        for c in range(GB):
            rows = rows_a if c % 2 == 0 else rows_b
            sem = sem_a if c % 2 == 0 else sem_b
            _wait(_i32(c), rows, sem)
            _scale(rows, _i32(c))
            _scatter(rows, _i32(c))
            if c + 2 < GB:
                _gather(_i32(c + 2), rows, sem)
        return carry
    lax.fori_loop(_i32(0), _i32(NBLK), _block, _i32(0))

    pltpu.sync_copy(den_v, den_hbm.at[wid])
    plsc.subcore_barrier()
    for off, ln in _RPT_CHUNKS:
        sl = pl.ds(row0 + _i32(off), ln)
        pltpu.sync_copy(acc_sh.at[sl], out_hbm.at[cid, sl])


_sc_call = pl.kernel(
    _sc_body,
    out_type=[
        jax.ShapeDtypeStruct((NC, NPAD, F), jnp.float32),
        jax.ShapeDtypeStruct((NW, NPAD), jnp.float32),
    ],
    mesh=plsc.VectorSubcoreMesh(
        core_axis_name="c", subcore_axis_name="s",
        num_cores=NC, num_subcores=NS),
    scratch_types=[
        pltpu.VMEM((NPAD,), jnp.float32),
        pltpu.VMEM((NPAD,), jnp.float32),
        pltpu.VMEM((NPAD,), jnp.float32),
        pltpu.VMEM((GB, CH), jnp.int32),
        pltpu.VMEM((GB, CH), jnp.int32),
        pltpu.VMEM((GB, CH), jnp.float32),
        pltpu.VMEM((CH, F), jnp.float32),
        pltpu.VMEM((CH, F), jnp.float32),
        pltpu.VMEM_SHARED((NPAD, F), jnp.float32),
        pltpu.SemaphoreType.DMA,
        pltpu.SemaphoreType.DMA,
    ],
    compiler_params=pltpu.CompilerParams(needs_layout_passes=False),
)


def _tc_post_body(p0_ref, p1_ref, den_ref, b_ref, o_ref):
    den = jnp.sum(den_ref[...], axis=0) + jnp.float32(1e-16)
    o_ref[...] = (p0_ref[...] + p1_ref[...]) / den[:, None] + b_ref[...]


_tc_post = pl.pallas_call(
    _tc_post_body,
    grid=(NPAD // TCB,),
    in_specs=[
        pl.BlockSpec((TCB, F), lambda i: (i, i * 0)),
        pl.BlockSpec((TCB, F), lambda i: (i, i * 0)),
        pl.BlockSpec((NW, TCB), lambda i: (i * 0, i)),
        pl.BlockSpec((1, F), lambda i: (i * 0, i * 0)),
    ],
    out_specs=pl.BlockSpec((TCB, F), lambda i: (i, i * 0)),
    out_shape=jax.ShapeDtypeStruct((NPAD, F), jnp.float32),
)


def kernel(x, edge_index, weight, att, bias):
    x = x.astype(jnp.float32)
    weight = weight.astype(jnp.float32)
    att_f = att.astype(jnp.float32).reshape(2 * F)
    bias = bias.astype(jnp.float32)

    src = edge_index[0].astype(jnp.int32)
    dst = edge_index[1].astype(jnp.int32)
    dst = jnp.where(src == dst, jnp.int32(N), dst)
    loops = jnp.arange(N, dtype=jnp.int32)
    pad_src = jnp.arange(EPAD - E2, dtype=jnp.int32) % jnp.int32(N)
    pad_dst = jnp.int32(N + 16) + (jnp.arange(EPAD - E2, dtype=jnp.int32)
                                   % jnp.int32(NPAD - N - 16))
    src2 = jnp.concatenate([src, loops, pad_src]).reshape(NW, NBLK, GB, CH)
    dst2 = jnp.concatenate([dst, loops, pad_dst]).reshape(NW, NBLK, GB, CH)

    x_pad = jnp.zeros((NPAD, F), jnp.float32).at[:N].set(x)
    attmat = (jnp.zeros((F, F), jnp.float32)
              .at[:, 0].set(att_f[:F])
              .at[:, 1].set(att_f[F:]))

    h, s = _tc_pre(x_pad, weight, attmat)
    ai = s[:, 0] + 0.0
    aj = s[:, 1] + 0.0

    partials, dens = _sc_call(src2, dst2, h, ai, aj)

    out = _tc_post(partials[0], partials[1], dens, bias.reshape(1, F))
    return out[:N]


# speedup vs baseline: 1.1892x; 1.0189x over previous
import jax
import jax.numpy as jnp
from jax import lax
from jax.experimental import pallas as pl
from jax.experimental.pallas import tpu as pltpu
from jax.experimental.pallas import tpu_sc as plsc

N = 10000
F = 128
NEG_SLOPE = 0.2

NC = 2
NS = 16
L = 16
NW = NC * NS

NPAD = 10112
RPT = NPAD // NS
CH = 64
GB = 8
NBLK = 21
NCHUNK = GB * NBLK
EW = CH * NCHUNK
EPAD = EW * NW
E2 = 320000 + N

TCB = NPAD

_RPT_CHUNKS = []
_off = 0
while _off < RPT:
    _ln = min(64, RPT - _off)
    _RPT_CHUNKS.append((_off, _ln))
    _off += _ln


def _i32(v):
    return jnp.int32(v)


def _tc_pre_body(x_ref, w_ref, am_ref, h_ref, s_ref):
    h = jnp.dot(x_ref[...], w_ref[...], preferred_element_type=jnp.float32)
    h_ref[...] = h
    s_ref[...] = jnp.dot(h, am_ref[...], preferred_element_type=jnp.float32)


_tc_pre = pl.pallas_call(
    _tc_pre_body,
    grid=(NPAD // TCB,),
    in_specs=[
        pl.BlockSpec((TCB, F), lambda i: (i, i * 0)),
        pl.BlockSpec((F, F), lambda i: (i * 0, i * 0)),
        pl.BlockSpec((F, F), lambda i: (i * 0, i * 0)),
    ],
    out_specs=[
        pl.BlockSpec((TCB, F), lambda i: (i, i * 0)),
        pl.BlockSpec((TCB, F), lambda i: (i, i * 0)),
    ],
    out_shape=[
        jax.ShapeDtypeStruct((NPAD, F), jnp.float32),
        jax.ShapeDtypeStruct((NPAD, F), jnp.float32),
    ],
)


def _sc_body(src_hbm, dst_hbm, h_hbm, ai_hbm, aj_hbm,
             out_hbm, den_hbm,
             ai_v, aj_v, den_v, src_v, dst_v, w_v,
             rows_a, rows_b, acc_sh, sem_a, sem_b):
    cid = lax.axis_index("c").astype(jnp.int32)
    sid = lax.axis_index("s").astype(jnp.int32)
    wid = sid * _i32(NC) + cid

    pltpu.sync_copy(ai_hbm, ai_v)
    pltpu.sync_copy(aj_hbm, aj_v)

    zero16 = jnp.zeros((L,), jnp.float32)

    def _zero_den(i, carry):
        den_v[pl.ds(i * _i32(L), L)] = zero16
        return carry
    lax.fori_loop(_i32(0), _i32(NPAD // L), _zero_den, _i32(0))

    def _zero_rows(r, carry):
        for c in range(F // L):
            rows_a[r, pl.ds(c * L, L)] = zero16
        return carry
    lax.fori_loop(_i32(0), _i32(CH), _zero_rows, _i32(0))

    row0 = sid * _i32(RPT)
    for off, ln in _RPT_CHUNKS:
        pltpu.sync_copy(rows_a.at[pl.ds(0, ln)],
                        acc_sh.at[pl.ds(row0 + _i32(off), ln)])
    plsc.subcore_barrier()

    def _gather(c, rows, sem):
        pltpu.async_copy(h_hbm.at[src_v.at[c]], rows, sem)

    def _wait(c, rows, sem):
        pltpu.make_async_copy(h_hbm.at[src_v.at[c]], rows, sem).wait()

    def _scale(rows, c):
        def body(r4, carry):
            cidx = jnp.full((L,), c, jnp.int32)
            base = r4 * _i32(4)
            rr = [base + _i32(k) for k in range(4)]
            wbs = [plsc.load_gather(w_v, [cidx, jnp.full((L,), r, jnp.int32)])
                   for r in rr]
            for q in range(F // L):
                for r, wb in zip(rr, wbs):
                    rows[r, pl.ds(q * L, L)] = rows[r, pl.ds(q * L, L)] * wb
            return carry
        lax.fori_loop(_i32(0), _i32(CH // 4), body, _i32(0))

    def _scatter(rows, c):
        pltpu.sync_copy(rows, acc_sh.at[dst_v.at[c]], add=True)

    def _block(b, carry):
        pltpu.sync_copy(src_hbm.at[wid, b], src_v)
        pltpu.sync_copy(dst_hbm.at[wid, b], dst_v)

        _gather(_i32(0), rows_a, sem_a)
        _gather(_i32(1), rows_b, sem_b)

        def _alpha(j, c2):
            r = lax.shift_right_logical(j, _i32(2))
            col = lax.shift_left(jnp.bitwise_and(j, _i32(3)), _i32(4))
            s16 = src_v[r, pl.ds(col, L)]
            d16 = dst_v[r, pl.ds(col, L)]
            a = plsc.load_gather(ai_v, [d16]) + plsc.load_gather(aj_v, [s16])
            a = jnp.where(a >= 0, a, a * NEG_SLOPE)
            wv = jnp.exp(a)
            w_v[r, pl.ds(col, L)] = wv
            plsc.addupdate_scatter(den_v, [d16], wv)
            return c2
        lax.fori_loop(_i32(0), _i32(GB * CH // L), _alpha, _i32(0))

        for c in range(GB):
            rows = rows_a if c % 2 == 0 else rows_b
            sem = sem_a if c % 2 == 0 else sem_b
            _wait(_i32(c), rows, sem)
            _scale(rows, _i32(c))
            _scatter(rows, _i32(c))
            if c + 2 < GB:
                _gather(_i32(c + 2), rows, sem)
        return carry
    lax.fori_loop(_i32(0), _i32(NBLK), _block, _i32(0))

    pltpu.sync_copy(den_v, den_hbm.at[wid])
    plsc.subcore_barrier()
    for off, ln in _RPT_CHUNKS:
        sl = pl.ds(row0 + _i32(off), ln)
        pltpu.sync_copy(acc_sh.at[sl], out_hbm.at[cid, sl])


_sc_call = pl.kernel(
    _sc_body,
    out_type=[
        jax.ShapeDtypeStruct((NC, NPAD, F), jnp.float32),
        jax.ShapeDtypeStruct((NW, NPAD), jnp.float32),
    ],
    mesh=plsc.VectorSubcoreMesh(
        core_axis_name="c", subcore_axis_name="s",
        num_cores=NC, num_subcores=NS),
    scratch_types=[
        pltpu.VMEM((NPAD,), jnp.float32),
        pltpu.VMEM((NPAD,), jnp.float32),
        pltpu.VMEM((NPAD,), jnp.float32),
        pltpu.VMEM((GB, CH), jnp.int32),
        pltpu.VMEM((GB, CH), jnp.int32),
        pltpu.VMEM((GB, CH), jnp.float32),
        pltpu.VMEM((CH, F), jnp.float32),
        pltpu.VMEM((CH, F), jnp.float32),
        pltpu.VMEM_SHARED((NPAD, F), jnp.float32),
        pltpu.SemaphoreType.DMA,
        pltpu.SemaphoreType.DMA,
    ],
    compiler_params=pltpu.CompilerParams(needs_layout_passes=False),
)


def _tc_post_body(p0_ref, p1_ref, den_ref, b_ref, o_ref):
    den = jnp.sum(den_ref[...], axis=0) + jnp.float32(1e-16)
    o_ref[...] = (p0_ref[...] + p1_ref[...]) / den[:, None] + b_ref[...]


_tc_post = pl.pallas_call(
    _tc_post_body,
    grid=(NPAD // TCB,),
    in_specs=[
        pl.BlockSpec((TCB, F), lambda i: (i, i * 0)),
        pl.BlockSpec((TCB, F), lambda i: (i, i * 0)),
        pl.BlockSpec((NW, TCB), lambda i: (i * 0, i)),
        pl.BlockSpec((1, F), lambda i: (i * 0, i * 0)),
    ],
    out_specs=pl.BlockSpec((TCB, F), lambda i: (i, i * 0)),
    out_shape=jax.ShapeDtypeStruct((NPAD, F), jnp.float32),
)


def kernel(x, edge_index, weight, att, bias):
    x = x.astype(jnp.float32)
    weight = weight.astype(jnp.float32)
    att_f = att.astype(jnp.float32).reshape(2 * F)
    bias = bias.astype(jnp.float32)

    src = edge_index[0].astype(jnp.int32)
    dst = edge_index[1].astype(jnp.int32)
    dst = jnp.where(src == dst, jnp.int32(N), dst)
    loops = jnp.arange(N, dtype=jnp.int32)
    pad_src = jnp.arange(EPAD - E2, dtype=jnp.int32) % jnp.int32(N)
    pad_dst = jnp.int32(N + 16) + (jnp.arange(EPAD - E2, dtype=jnp.int32)
                                   % jnp.int32(NPAD - N - 16))
    src2 = jnp.concatenate([src, loops, pad_src]).reshape(NW, NBLK, GB, CH)
    dst2 = jnp.concatenate([dst, loops, pad_dst]).reshape(NW, NBLK, GB, CH)

    x_pad = jnp.zeros((NPAD, F), jnp.float32).at[:N].set(x)
    attmat = (jnp.zeros((F, F), jnp.float32)
              .at[:, 0].set(att_f[:F])
              .at[:, 1].set(att_f[F:]))

    h, s = _tc_pre(x_pad, weight, attmat)
    ai = s[:, 0] + 0.0
    aj = s[:, 1] + 0.0

    partials, dens = _sc_call(src2, dst2, h, ai, aj)

    out = _tc_post(partials[0], partials[1], dens, bias.reshape(1, F))
    return out[:N]
